# baseline (speedup 1.0000x reference)
import functools

import jax
import jax.numpy as jnp
from jax import lax
from jax.experimental import pallas as pl
from jax.experimental.pallas import tpu as pltpu
from jax.experimental.pallas import tpu_sc as plsc

BATCH = 4096
CTX = 50
VOCAB = 1000
HIDDEN = 128

NC = 2
NS = 16
NW = NC * NS
BPW = BATCH // NW
NCHUNK = CTX
NBUF = 7
NFULL = (NCHUNK - 2 * NBUF) // NBUF
NREM = (NCHUNK - 2 * NBUF) % NBUF


@functools.partial(
    pl.kernel,
    out_type=jax.ShapeDtypeStruct((CTX, BATCH, HIDDEN), jnp.float32),
    mesh=plsc.VectorSubcoreMesh(core_axis_name="c", subcore_axis_name="s"),
    scratch_types=[
        pltpu.VMEM((CTX, BPW), jnp.int32),
        pltpu.VMEM((NBUF, BPW, HIDDEN), jnp.float32),
        pltpu.VMEM_SHARED((VOCAB, HIDDEN), jnp.float32),
    ] + [pltpu.SemaphoreType.DMA] * (2 * NBUF + 1),
)
def _gather_kernel(labels_t_hbm, table_hbm, out_hbm, idx_v, rows_v,
                   table_s, g0, g1, g2, g3, g4, g5, g6,
                   o0, o1, o2, o3, o4, o5, o6, tsem):
    gsem = (g0, g1, g2, g3, g4, g5, g6)
    osem = (o0, o1, o2, o3, o4, o5, o6)
    sid = lax.axis_index("s")
    wid = sid * NC + lax.axis_index("c")
    base = wid * BPW

    @pl.when(sid < 13)
    def _stage_lo():
        r0 = sid * 64
        pltpu.async_copy(table_hbm.at[pl.ds(r0, 64)],
                         table_s.at[pl.ds(r0, 64)], tsem)

    @pl.when(sid >= 13)
    def _stage_hi():
        r0 = 832 + (sid - 13) * 56
        pltpu.async_copy(table_hbm.at[pl.ds(r0, 56)],
                         table_s.at[pl.ds(r0, 56)], tsem)

    pltpu.sync_copy(labels_t_hbm.at[:, pl.ds(base, BPW)], idx_v)

    @pl.when(sid < 13)
    def _wait_lo():
        r0 = sid * 64
        pltpu.make_async_copy(table_hbm.at[pl.ds(r0, 64)],
                              table_s.at[pl.ds(r0, 64)], tsem).wait()

    @pl.when(sid >= 13)
    def _wait_hi():
        r0 = 832 + (sid - 13) * 56
        pltpu.make_async_copy(table_hbm.at[pl.ds(r0, 56)],
                              table_s.at[pl.ds(r0, 56)], tsem).wait()

    plsc.subcore_barrier()

    def start_gather(t, b):
        pltpu.async_copy(table_s.at[idx_v.at[t]], rows_v.at[b], gsem[b])

    def start_ocopy(t, b):
        pltpu.async_copy(rows_v.at[b], out_hbm.at[t, pl.ds(base, BPW)],
                         osem[b])

    def drain(b):
        pltpu.make_async_copy(
            rows_v.at[b], out_hbm.at[0, pl.ds(base, BPW)], osem[b]).wait()

    def wait_gather(b):
        pltpu.make_async_copy(table_s.at[idx_v.at[0]], rows_v.at[b],
                              gsem[b]).wait()

    start_gather(0, 0)
    for b in range(NBUF):
        t = b
        if t == NBUF - 1:
            drain((t + 1) % NBUF)
        start_gather(t + 1, (t + 1) % NBUF)
        wait_gather(t % NBUF)
        start_ocopy(t, t % NBUF)

    @pl.loop(NBUF, NBUF + NFULL * NBUF, step=NBUF)
    def _block(j):
        for b in range(NBUF):
            t = j + b
            bb = (b + 1) % NBUF
            drain(bb)
            start_gather(t + 1, bb)
            wait_gather(b)
            start_ocopy(t, b)

    for k in range(NREM):
        t = NBUF + NFULL * NBUF + k
        b = t % NBUF
        bb = (b + 1) % NBUF
        drain(bb)
        start_gather(t + 1, bb)
        wait_gather(b)
        start_ocopy(t, b)

    for k in range(NBUF):
        t = NCHUNK - NBUF + k
        b = t % NBUF
        if k < NBUF - 1:
            bb = (b + 1) % NBUF
            drain(bb)
            start_gather(t + 1, bb)
        wait_gather(b)
        start_ocopy(t, b)
    for b in range(NBUF):
        drain(b)


def kernel(labels, class_embedding):
    labels_t = labels.astype(jnp.int32).T
    out_cbh = _gather_kernel(labels_t, class_embedding)
    return out_cbh.transpose(1, 0, 2)


# speedup vs baseline: 1.0028x; 1.0028x over previous
import functools

import jax
import jax.numpy as jnp
from jax import lax
from jax.experimental import pallas as pl
from jax.experimental.pallas import tpu as pltpu
from jax.experimental.pallas import tpu_sc as plsc

BATCH = 4096
CTX = 50
VOCAB = 1000
HIDDEN = 128

NC = 2
NS = 16
NW = NC * NS
BPW = BATCH // NW
NCHUNK = CTX
NBUF = 5


@functools.partial(
    pl.kernel,
    out_type=jax.ShapeDtypeStruct((CTX, BATCH, HIDDEN), jnp.float32),
    mesh=plsc.VectorSubcoreMesh(core_axis_name="c", subcore_axis_name="s"),
    scratch_types=[
        pltpu.VMEM((CTX, BPW), jnp.int32),
        pltpu.VMEM((NBUF, BPW, HIDDEN), jnp.float32),
        pltpu.VMEM_SHARED((VOCAB, HIDDEN), jnp.float32),
    ] + [pltpu.SemaphoreType.DMA] * (2 * NBUF + 1),
)
def _gather_kernel(labels_t_hbm, table_hbm, out_hbm, idx_v, rows_v,
                   table_s, g0, g1, g2, g3, g4, o0, o1, o2, o3, o4, tsem):
    gsem = (g0, g1, g2, g3, g4)
    osem = (o0, o1, o2, o3, o4)
    sid = lax.axis_index("s")
    wid = sid * NC + lax.axis_index("c")
    base = wid * BPW

    @pl.when(sid < 13)
    def _stage_lo():
        r0 = sid * 64
        pltpu.async_copy(table_hbm.at[pl.ds(r0, 64)],
                         table_s.at[pl.ds(r0, 64)], tsem)

    @pl.when(sid >= 13)
    def _stage_hi():
        r0 = 832 + (sid - 13) * 56
        pltpu.async_copy(table_hbm.at[pl.ds(r0, 56)],
                         table_s.at[pl.ds(r0, 56)], tsem)

    pltpu.sync_copy(labels_t_hbm.at[:, pl.ds(base, BPW)], idx_v)

    @pl.when(sid < 13)
    def _wait_lo():
        r0 = sid * 64
        pltpu.make_async_copy(table_hbm.at[pl.ds(r0, 64)],
                              table_s.at[pl.ds(r0, 64)], tsem).wait()

    @pl.when(sid >= 13)
    def _wait_hi():
        r0 = 832 + (sid - 13) * 56
        pltpu.make_async_copy(table_hbm.at[pl.ds(r0, 56)],
                              table_s.at[pl.ds(r0, 56)], tsem).wait()

    plsc.subcore_barrier()

    def start_gather(t, b):
        pltpu.async_copy(table_s.at[idx_v.at[t]], rows_v.at[b], gsem[b])

    def start_ocopy(t, b):
        pltpu.async_copy(rows_v.at[b], out_hbm.at[t, pl.ds(base, BPW)],
                         osem[b])

    def drain(b):
        pltpu.make_async_copy(
            rows_v.at[b], out_hbm.at[0, pl.ds(base, BPW)], osem[b]).wait()

    def wait_gather(b):
        pltpu.make_async_copy(table_s.at[idx_v.at[0]], rows_v.at[b],
                              gsem[b]).wait()

    start_gather(0, 0)
    for b in range(NBUF):
        t = b
        if t == NBUF - 1:
            drain((t + 1) % NBUF)
        start_gather(t + 1, (t + 1) % NBUF)
        wait_gather(t % NBUF)
        start_ocopy(t, t % NBUF)

    @pl.loop(NBUF, NCHUNK - NBUF, step=NBUF)
    def _block(j):
        for b in range(NBUF):
            t = j + b
            bb = (b + 1) % NBUF
            drain(bb)
            start_gather(t + 1, bb)
            wait_gather(b)
            start_ocopy(t, b)

    for b in range(NBUF):
        t = NCHUNK - NBUF + b
        if b < NBUF - 1:
            drain((b + 1) % NBUF)
            start_gather(t + 1, (b + 1) % NBUF)
        wait_gather(b)
        start_ocopy(t, b)
    for b in range(NBUF):
        drain(b)


def kernel(labels, class_embedding):
    labels_t = labels.astype(jnp.int32).T
    out_cbh = _gather_kernel(labels_t, class_embedding)
    return out_cbh.transpose(1, 0, 2)
